# baseline (speedup 1.0000x reference)
import jax
import jax.numpy as jnp
from jax.experimental import pallas as pl
from jax.experimental.pallas import tpu as pltpu

N = 10000
NFEAT = 128
NHID = 16
OUT = 64
NCLASS = 10
NLAYERS = 9
FIN = NFEAT + 2 * NHID * NLAYERS

RB_FIRST = 200
RB_ADJ = 400
RB_ATT = 1000


def _combine_w(wa, wb, k):
    ca, cb = wa.shape[1], wb.shape[1]
    parts = [jnp.concatenate([wa[:NFEAT], wb[:NFEAT]], axis=1)]
    za = jnp.zeros((NHID, cb), jnp.float32)
    zb = jnp.zeros((NHID, ca), jnp.float32)
    for j in range(k):
        ra = wa[NFEAT + NHID * j:NFEAT + NHID * (j + 1)]
        rb = wb[NFEAT + NHID * j:NFEAT + NHID * (j + 1)]
        parts.append(jnp.concatenate([ra, za], axis=1))
        parts.append(jnp.concatenate([zb, rb], axis=1))
    fin = NFEAT + 2 * NHID * k
    if fin < FIN:
        parts.append(jnp.zeros((FIN - fin, ca + cb), jnp.float32))
    return jnp.concatenate(parts, axis=0).astype(jnp.bfloat16)


def _project(inp_ref, w_ref, xw_ref):
    @pl.when(pl.program_id(0) == 0)
    def _():
        xw_ref[...] = jnp.dot(inp_ref[...], w_ref[...],
                              preferred_element_type=jnp.float32
                              ).astype(jnp.bfloat16)


def _adj_first_body(adj_ref, inp_ref, w_ref, b_ref, h_ref, adjb_ref, xw_ref):
    _project(inp_ref, w_ref, xw_ref)
    a = adj_ref[...].astype(jnp.bfloat16)
    adjb_ref[...] = a
    y = jnp.dot(a, xw_ref[...], preferred_element_type=jnp.float32)
    h_ref[...] = jnp.tanh(y + b_ref[...]).astype(jnp.bfloat16)


def _adj_first_call(adj, inp, w, b):
    width = w.shape[1]
    return pl.pallas_call(
        _adj_first_body,
        grid=(N // RB_FIRST,),
        in_specs=[
            pl.BlockSpec((RB_FIRST, N), lambda i: (i, 0)),
            pl.BlockSpec((N, FIN), lambda i: (0, 0)),
            pl.BlockSpec((FIN, width), lambda i: (0, 0)),
            pl.BlockSpec((1, width), lambda i: (0, 0)),
        ],
        out_specs=[
            pl.BlockSpec((RB_FIRST, width), lambda i: (i, 0)),
            pl.BlockSpec((RB_FIRST, N), lambda i: (i, 0)),
        ],
        out_shape=[
            jax.ShapeDtypeStruct((N, width), jnp.bfloat16),
            jax.ShapeDtypeStruct((N, N), jnp.bfloat16),
        ],
        scratch_shapes=[pltpu.VMEM((N, width), jnp.bfloat16)],
    )(adj, inp, w, b)


def _adj_mid_body(adj_ref, inp_ref, w_ref, b_ref, h_ref, xw_ref):
    _project(inp_ref, w_ref, xw_ref)
    y = jnp.dot(adj_ref[...], xw_ref[...], preferred_element_type=jnp.float32)
    h_ref[...] = jnp.tanh(y + b_ref[...]).astype(jnp.bfloat16)


def _adj_mid_call(adj_b, inp, w, b):
    width = w.shape[1]
    return pl.pallas_call(
        _adj_mid_body,
        grid=(N // RB_ADJ,),
        in_specs=[
            pl.BlockSpec((RB_ADJ, N), lambda i: (i, 0)),
            pl.BlockSpec((N, FIN), lambda i: (0, 0)),
            pl.BlockSpec((FIN, width), lambda i: (0, 0)),
            pl.BlockSpec((1, width), lambda i: (0, 0)),
        ],
        out_specs=pl.BlockSpec((RB_ADJ, width), lambda i: (i, 0)),
        out_shape=jax.ShapeDtypeStruct((N, width), jnp.bfloat16),
        scratch_shapes=[pltpu.VMEM((N, width), jnp.bfloat16)],
    )(adj_b, inp, w, b)


def _adj_final_body(adj_ref, inp_ref, w_ref, b_ref, oa_ref, ob_ref, z_ref):
    _project(inp_ref, w_ref, z_ref)
    y = jnp.dot(adj_ref[...], z_ref[...], preferred_element_type=jnp.float32)
    y = y + b_ref[...]
    ya = y[:, :OUT]
    yb = y[:, OUT:]
    na = jnp.maximum(jnp.sqrt(jnp.sum(ya * ya, axis=1, keepdims=True)), 1e-12)
    nb = jnp.maximum(jnp.sqrt(jnp.sum(yb * yb, axis=1, keepdims=True)), 1e-12)
    oa_ref[...] = ya / na
    ob_ref[...] = yb / nb


def _adj_final_call(adj_b, inp, w, b):
    width = w.shape[1]
    return pl.pallas_call(
        _adj_final_body,
        grid=(N // RB_ADJ,),
        in_specs=[
            pl.BlockSpec((RB_ADJ, N), lambda i: (i, 0)),
            pl.BlockSpec((N, FIN), lambda i: (0, 0)),
            pl.BlockSpec((FIN, width), lambda i: (0, 0)),
            pl.BlockSpec((1, width), lambda i: (0, 0)),
        ],
        out_specs=[
            pl.BlockSpec((RB_ADJ, OUT), lambda i: (i, 0)),
            pl.BlockSpec((RB_ADJ, OUT), lambda i: (i, 0)),
        ],
        out_shape=[
            jax.ShapeDtypeStruct((N, OUT), jnp.float32),
            jax.ShapeDtypeStruct((N, OUT), jnp.float32),
        ],
        scratch_shapes=[pltpu.VMEM((N, width), jnp.bfloat16)],
    )(adj_b, inp, w, b)


def _snowball_pair(x_pad, adj, pa, pb):
    inp = x_pad
    adj_b = None
    for k in range(NLAYERS):
        wc = _combine_w(pa["Ws"][k], pb["Ws"][k], k)
        bc = jnp.concatenate([pa["bs"][k], pb["bs"][k]]).reshape(1, 2 * NHID)
        if k == 0:
            h, adj_b = _adj_first_call(adj, inp, wc, bc)
        else:
            h = _adj_mid_call(adj_b, inp, wc, bc)
        inp = jax.lax.dynamic_update_slice(inp, h, (0, NFEAT + 2 * NHID * k))
    wc = _combine_w(pa["Wout"], pb["Wout"], NLAYERS)
    bc = jnp.concatenate([pa["bout"], pb["bout"]]).reshape(1, 2 * OUT)
    return _adj_final_call(adj_b, inp, wc, bc)


def _att_body(e1_ref, e2_ref, e3_ref, c1_ref, c2_ref, c3_ref,
              w1_ref, b1_ref, w2_ref, mw_ref, mb_ref, out_ref, beta_ref):
    xcom = (c1_ref[...] + c2_ref[...] + c3_ref[...]) / 3.0
    embs = (e1_ref[...], e2_ref[...], e3_ref[...], xcom)
    w2 = w2_ref[...]
    cols = []
    for e in embs:
        t = jnp.tanh(jnp.dot(e, w1_ref[...],
                             preferred_element_type=jnp.float32) + b1_ref[...])
        cols.append(t[:, 0:1] * w2[0:1, 0:1] + t[:, 1:2] * w2[1:2, 0:1])
    w = jnp.concatenate(cols, axis=1)
    m = jnp.max(w, axis=1, keepdims=True)
    ew = jnp.exp(w - m)
    beta = ew / jnp.sum(ew, axis=1, keepdims=True)
    beta_ref[...] = beta
    emb_att = (beta[:, 0:1] * embs[0] + beta[:, 1:2] * embs[1]
               + beta[:, 2:3] * embs[2] + beta[:, 3:4] * embs[3])
    logits = jnp.dot(emb_att, mw_ref[...],
                     preferred_element_type=jnp.float32) + mb_ref[...]
    mm = jnp.max(logits, axis=1, keepdims=True)
    el = jnp.exp(logits - mm)
    out_ref[...] = el / jnp.sum(el, axis=1, keepdims=True)


def _att_call(e1, e2, e3, c1, c2, c3, att_w1, att_b1, att_w2, mlp_w, mlp_b):
    emb_spec = pl.BlockSpec((RB_ATT, OUT), lambda i: (i, 0))
    full = lambda shape: pl.BlockSpec(shape, lambda i: (0, 0))
    return pl.pallas_call(
        _att_body,
        grid=(N // RB_ATT,),
        in_specs=[
            emb_spec, emb_spec, emb_spec, emb_spec, emb_spec, emb_spec,
            full((OUT, 2)), full((1, 2)), full((2, 1)),
            full((OUT, NCLASS)), full((1, NCLASS)),
        ],
        out_specs=[
            pl.BlockSpec((RB_ATT, NCLASS), lambda i: (i, 0)),
            pl.BlockSpec((RB_ATT, 4), lambda i: (i, 0)),
        ],
        out_shape=[
            jax.ShapeDtypeStruct((N, NCLASS), jnp.float32),
            jax.ShapeDtypeStruct((N, 4), jnp.float32),
        ],
    )(e1, e2, e3, c1, c2, c3, att_w1, att_b1.reshape(1, 2), att_w2,
      mlp_w, mlp_b.reshape(1, NCLASS))


def kernel(x, sadj, fadj, fadj2, sgcn1, sgcn2, sgcn3, cgcn,
           att_w1, att_b1, att_w2, mlp_w, mlp_b):
    x_pad = jnp.pad(x.astype(jnp.bfloat16), ((0, 0), (0, FIN - NFEAT)))
    emb1, com1 = _snowball_pair(x_pad, sadj, sgcn1, cgcn)
    emb2, com2 = _snowball_pair(x_pad, fadj, sgcn2, cgcn)
    emb3, com3 = _snowball_pair(x_pad, fadj2, sgcn3, cgcn)
    output, beta4 = _att_call(emb1, emb2, emb3, com1, com2, com3,
                              att_w1, att_b1, att_w2, mlp_w, mlp_b)
    beta = beta4.reshape(N, 4, 1)
    return (output, beta, emb1, com1, com2, com3, emb2, emb3)


# speedup vs baseline: 1.0432x; 1.0432x over previous
import jax
import jax.numpy as jnp
from jax.experimental import pallas as pl
from jax.experimental.pallas import tpu as pltpu

N = 10000
NFEAT = 128
NHID = 16
OUT = 64
NCLASS = 10
NLAYERS = 9
FIN = NFEAT + 2 * NHID * NLAYERS

RB_FIRST = 200
RB_ADJ = 1000
RB_ATT = 1000


def _combine_w(wa, wb, k):
    ca, cb = wa.shape[1], wb.shape[1]
    parts = [jnp.concatenate([wa[:NFEAT], wb[:NFEAT]], axis=1)]
    za = jnp.zeros((NHID, cb), jnp.float32)
    zb = jnp.zeros((NHID, ca), jnp.float32)
    for j in range(k):
        ra = wa[NFEAT + NHID * j:NFEAT + NHID * (j + 1)]
        rb = wb[NFEAT + NHID * j:NFEAT + NHID * (j + 1)]
        parts.append(jnp.concatenate([ra, za], axis=1))
        parts.append(jnp.concatenate([zb, rb], axis=1))
    fin = NFEAT + 2 * NHID * k
    if fin < FIN:
        parts.append(jnp.zeros((FIN - fin, ca + cb), jnp.float32))
    return jnp.concatenate(parts, axis=0).astype(jnp.bfloat16)


def _project(inp_ref, w_ref, xw_ref):
    @pl.when(pl.program_id(0) == 0)
    def _():
        xw_ref[...] = jnp.dot(inp_ref[...], w_ref[...],
                              preferred_element_type=jnp.float32
                              ).astype(jnp.bfloat16)


def _adj_first_body(adj_ref, inp_ref, w_ref, b_ref, h_ref, adjb_ref, xw_ref):
    _project(inp_ref, w_ref, xw_ref)
    a = adj_ref[...].astype(jnp.bfloat16)
    adjb_ref[...] = a
    y = jnp.dot(a, xw_ref[...], preferred_element_type=jnp.float32)
    h_ref[...] = jnp.tanh(y + b_ref[...]).astype(jnp.bfloat16)


def _adj_first_call(adj, inp, w, b):
    width = w.shape[1]
    return pl.pallas_call(
        _adj_first_body,
        grid=(N // RB_FIRST,),
        in_specs=[
            pl.BlockSpec((RB_FIRST, N), lambda i: (i, 0)),
            pl.BlockSpec((N, FIN), lambda i: (0, 0)),
            pl.BlockSpec((FIN, width), lambda i: (0, 0)),
            pl.BlockSpec((1, width), lambda i: (0, 0)),
        ],
        out_specs=[
            pl.BlockSpec((RB_FIRST, width), lambda i: (i, 0)),
            pl.BlockSpec((RB_FIRST, N), lambda i: (i, 0)),
        ],
        out_shape=[
            jax.ShapeDtypeStruct((N, width), jnp.bfloat16),
            jax.ShapeDtypeStruct((N, N), jnp.bfloat16),
        ],
        scratch_shapes=[pltpu.VMEM((N, width), jnp.bfloat16)],
    )(adj, inp, w, b)


def _adj_mid_body(adj_ref, inp_ref, w_ref, b_ref, h_ref, xw_ref):
    _project(inp_ref, w_ref, xw_ref)
    y = jnp.dot(adj_ref[...], xw_ref[...], preferred_element_type=jnp.float32)
    h_ref[...] = jnp.tanh(y + b_ref[...]).astype(jnp.bfloat16)


def _adj_mid_call(adj_b, inp, w, b):
    width = w.shape[1]
    return pl.pallas_call(
        _adj_mid_body,
        grid=(N // RB_ADJ,),
        in_specs=[
            pl.BlockSpec((RB_ADJ, N), lambda i: (i, 0)),
            pl.BlockSpec((N, FIN), lambda i: (0, 0)),
            pl.BlockSpec((FIN, width), lambda i: (0, 0)),
            pl.BlockSpec((1, width), lambda i: (0, 0)),
        ],
        out_specs=pl.BlockSpec((RB_ADJ, width), lambda i: (i, 0)),
        out_shape=jax.ShapeDtypeStruct((N, width), jnp.bfloat16),
        scratch_shapes=[pltpu.VMEM((N, width), jnp.bfloat16)],
    )(adj_b, inp, w, b)


def _adj_final_body(adj_ref, inp_ref, w_ref, b_ref, oa_ref, ob_ref, z_ref):
    _project(inp_ref, w_ref, z_ref)
    y = jnp.dot(adj_ref[...], z_ref[...], preferred_element_type=jnp.float32)
    y = y + b_ref[...]
    ya = y[:, :OUT]
    yb = y[:, OUT:]
    na = jnp.maximum(jnp.sqrt(jnp.sum(ya * ya, axis=1, keepdims=True)), 1e-12)
    nb = jnp.maximum(jnp.sqrt(jnp.sum(yb * yb, axis=1, keepdims=True)), 1e-12)
    oa_ref[...] = ya / na
    ob_ref[...] = yb / nb


def _adj_final_call(adj_b, inp, w, b):
    width = w.shape[1]
    return pl.pallas_call(
        _adj_final_body,
        grid=(N // RB_ADJ,),
        in_specs=[
            pl.BlockSpec((RB_ADJ, N), lambda i: (i, 0)),
            pl.BlockSpec((N, FIN), lambda i: (0, 0)),
            pl.BlockSpec((FIN, width), lambda i: (0, 0)),
            pl.BlockSpec((1, width), lambda i: (0, 0)),
        ],
        out_specs=[
            pl.BlockSpec((RB_ADJ, OUT), lambda i: (i, 0)),
            pl.BlockSpec((RB_ADJ, OUT), lambda i: (i, 0)),
        ],
        out_shape=[
            jax.ShapeDtypeStruct((N, OUT), jnp.float32),
            jax.ShapeDtypeStruct((N, OUT), jnp.float32),
        ],
        scratch_shapes=[pltpu.VMEM((N, width), jnp.bfloat16)],
    )(adj_b, inp, w, b)


def _snowball_pair(x_pad, adj, pa, pb):
    inp = x_pad
    adj_b = None
    for k in range(NLAYERS):
        wc = _combine_w(pa["Ws"][k], pb["Ws"][k], k)
        bc = jnp.concatenate([pa["bs"][k], pb["bs"][k]]).reshape(1, 2 * NHID)
        if k == 0:
            h, adj_b = _adj_first_call(adj, inp, wc, bc)
        else:
            h = _adj_mid_call(adj_b, inp, wc, bc)
        inp = jax.lax.dynamic_update_slice(inp, h, (0, NFEAT + 2 * NHID * k))
    wc = _combine_w(pa["Wout"], pb["Wout"], NLAYERS)
    bc = jnp.concatenate([pa["bout"], pb["bout"]]).reshape(1, 2 * OUT)
    return _adj_final_call(adj_b, inp, wc, bc)


def _att_body(e1_ref, e2_ref, e3_ref, c1_ref, c2_ref, c3_ref,
              w1_ref, b1_ref, w2_ref, mw_ref, mb_ref, out_ref, beta_ref):
    xcom = (c1_ref[...] + c2_ref[...] + c3_ref[...]) / 3.0
    embs = (e1_ref[...], e2_ref[...], e3_ref[...], xcom)
    w2 = w2_ref[...]
    cols = []
    for e in embs:
        t = jnp.tanh(jnp.dot(e, w1_ref[...],
                             preferred_element_type=jnp.float32) + b1_ref[...])
        cols.append(t[:, 0:1] * w2[0:1, 0:1] + t[:, 1:2] * w2[1:2, 0:1])
    w = jnp.concatenate(cols, axis=1)
    m = jnp.max(w, axis=1, keepdims=True)
    ew = jnp.exp(w - m)
    beta = ew / jnp.sum(ew, axis=1, keepdims=True)
    beta_ref[...] = beta
    emb_att = (beta[:, 0:1] * embs[0] + beta[:, 1:2] * embs[1]
               + beta[:, 2:3] * embs[2] + beta[:, 3:4] * embs[3])
    logits = jnp.dot(emb_att, mw_ref[...],
                     preferred_element_type=jnp.float32) + mb_ref[...]
    mm = jnp.max(logits, axis=1, keepdims=True)
    el = jnp.exp(logits - mm)
    out_ref[...] = el / jnp.sum(el, axis=1, keepdims=True)


def _att_call(e1, e2, e3, c1, c2, c3, att_w1, att_b1, att_w2, mlp_w, mlp_b):
    emb_spec = pl.BlockSpec((RB_ATT, OUT), lambda i: (i, 0))
    full = lambda shape: pl.BlockSpec(shape, lambda i: (0, 0))
    return pl.pallas_call(
        _att_body,
        grid=(N // RB_ATT,),
        in_specs=[
            emb_spec, emb_spec, emb_spec, emb_spec, emb_spec, emb_spec,
            full((OUT, 2)), full((1, 2)), full((2, 1)),
            full((OUT, NCLASS)), full((1, NCLASS)),
        ],
        out_specs=[
            pl.BlockSpec((RB_ATT, NCLASS), lambda i: (i, 0)),
            pl.BlockSpec((RB_ATT, 4), lambda i: (i, 0)),
        ],
        out_shape=[
            jax.ShapeDtypeStruct((N, NCLASS), jnp.float32),
            jax.ShapeDtypeStruct((N, 4), jnp.float32),
        ],
    )(e1, e2, e3, c1, c2, c3, att_w1, att_b1.reshape(1, 2), att_w2,
      mlp_w, mlp_b.reshape(1, NCLASS))


def kernel(x, sadj, fadj, fadj2, sgcn1, sgcn2, sgcn3, cgcn,
           att_w1, att_b1, att_w2, mlp_w, mlp_b):
    x_pad = jnp.pad(x.astype(jnp.bfloat16), ((0, 0), (0, FIN - NFEAT)))
    emb1, com1 = _snowball_pair(x_pad, sadj, sgcn1, cgcn)
    emb2, com2 = _snowball_pair(x_pad, fadj, sgcn2, cgcn)
    emb3, com3 = _snowball_pair(x_pad, fadj2, sgcn3, cgcn)
    output, beta4 = _att_call(emb1, emb2, emb3, com1, com2, com3,
                              att_w1, att_b1, att_w2, mlp_w, mlp_b)
    beta = beta4.reshape(N, 4, 1)
    return (output, beta, emb1, com1, com2, com3, emb2, emb3)


# speedup vs baseline: 1.0459x; 1.0026x over previous
import jax
import jax.numpy as jnp
from jax.experimental import pallas as pl
from jax.experimental.pallas import tpu as pltpu

N = 10000
NTOP = 5000
NFEAT = 128
NHID = 16
OUT = 64
NCLASS = 10
NLAYERS = 9
FIN = NFEAT + 2 * NHID * NLAYERS

RB = 200
NB = NTOP // RB
RB_ATT = 1000

_DEQ = 1.0 / 254.0


def _combine_w(wa, wb, k):
    ca, cb = wa.shape[1], wb.shape[1]
    parts = [jnp.concatenate([wa[:NFEAT], wb[:NFEAT]], axis=1)]
    za = jnp.zeros((NHID, cb), jnp.float32)
    zb = jnp.zeros((NHID, ca), jnp.float32)
    for j in range(k):
        ra = wa[NFEAT + NHID * j:NFEAT + NHID * (j + 1)]
        rb = wb[NFEAT + NHID * j:NFEAT + NHID * (j + 1)]
        parts.append(jnp.concatenate([ra, za], axis=1))
        parts.append(jnp.concatenate([zb, rb], axis=1))
    fin = NFEAT + 2 * NHID * k
    if fin < FIN:
        parts.append(jnp.zeros((FIN - fin, ca + cb), jnp.float32))
    return jnp.concatenate(parts, axis=0).astype(jnp.bfloat16)


def _project(inp_ref, w_ref, xw_ref):
    @pl.when(pl.program_id(0) == 0)
    def _():
        xw_ref[...] = jnp.dot(inp_ref[...], w_ref[...],
                              preferred_element_type=jnp.float32
                              ).astype(jnp.bfloat16)


def _dequant(q_ref):
    return (q_ref[...].astype(jnp.bfloat16) + 127.0) * _DEQ


_blk = lambda rb: pl.BlockSpec((rb, N), lambda i: (i, 0))
_blk_lo = lambda rb: pl.BlockSpec((rb, N), lambda i: (i + NB, 0))
_full = lambda shape: pl.BlockSpec(shape, lambda i: (0, 0))
_out_blk = lambda w: pl.BlockSpec((RB, w), lambda i: (i, 0))


def _adj_first_body(at_ref, ab_ref, inp_ref, w_ref, b_ref,
                    ht_ref, hb_ref, adjb_ref, adjq_ref, xw_ref):
    _project(inp_ref, w_ref, xw_ref)
    xw = xw_ref[...]
    b = b_ref[...]
    at = at_ref[...].astype(jnp.bfloat16)
    adjb_ref[...] = at
    ht_ref[...] = jnp.tanh(
        jnp.dot(at, xw, preferred_element_type=jnp.float32) + b
    ).astype(jnp.bfloat16)
    ab = ab_ref[...]
    adjq_ref[...] = jnp.clip(jnp.round(ab * 254.0 - 127.0),
                             -127.0, 127.0).astype(jnp.int8)
    yb = jnp.dot(ab.astype(jnp.bfloat16), xw,
                 preferred_element_type=jnp.float32)
    hb_ref[...] = jnp.tanh(yb + b).astype(jnp.bfloat16)


def _adj_first_call(adj, inp, w, b):
    width = w.shape[1]
    return pl.pallas_call(
        _adj_first_body,
        grid=(NB,),
        in_specs=[_blk(RB), _blk_lo(RB), _full((N, FIN)),
                  _full((FIN, width)), _full((1, width))],
        out_specs=[_out_blk(width), _out_blk(width), _blk(RB), _blk(RB)],
        out_shape=[
            jax.ShapeDtypeStruct((NTOP, width), jnp.bfloat16),
            jax.ShapeDtypeStruct((NTOP, width), jnp.bfloat16),
            jax.ShapeDtypeStruct((NTOP, N), jnp.bfloat16),
            jax.ShapeDtypeStruct((NTOP, N), jnp.int8),
        ],
        scratch_shapes=[pltpu.VMEM((N, width), jnp.bfloat16)],
    )(adj, adj, inp, w, b)


def _adj_mid_body(adjb_ref, adjq_ref, inp_ref, w_ref, b_ref,
                  ht_ref, hb_ref, xw_ref):
    _project(inp_ref, w_ref, xw_ref)
    xw = xw_ref[...]
    b = b_ref[...]
    yt = jnp.dot(adjb_ref[...], xw, preferred_element_type=jnp.float32)
    ht_ref[...] = jnp.tanh(yt + b).astype(jnp.bfloat16)
    yb = jnp.dot(_dequant(adjq_ref), xw, preferred_element_type=jnp.float32)
    hb_ref[...] = jnp.tanh(yb + b).astype(jnp.bfloat16)


def _adj_mid_call(adj_b, adj_q, inp, w, b):
    width = w.shape[1]
    return pl.pallas_call(
        _adj_mid_body,
        grid=(NB,),
        in_specs=[_blk(RB), _blk(RB), _full((N, FIN)),
                  _full((FIN, width)), _full((1, width))],
        out_specs=[_out_blk(width), _out_blk(width)],
        out_shape=[
            jax.ShapeDtypeStruct((NTOP, width), jnp.bfloat16),
            jax.ShapeDtypeStruct((NTOP, width), jnp.bfloat16),
        ],
        scratch_shapes=[pltpu.VMEM((N, width), jnp.bfloat16)],
    )(adj_b, adj_q, inp, w, b)


def _l2norm(y):
    n = jnp.maximum(jnp.sqrt(jnp.sum(y * y, axis=1, keepdims=True)), 1e-12)
    return y / n


def _adj_final_body(adjb_ref, adjq_ref, inp_ref, w_ref, b_ref,
                    oat_ref, obt_ref, oab_ref, obb_ref, z_ref):
    _project(inp_ref, w_ref, z_ref)
    z = z_ref[...]
    b = b_ref[...]
    yt = jnp.dot(adjb_ref[...], z, preferred_element_type=jnp.float32) + b
    oat_ref[...] = _l2norm(yt[:, :OUT])
    obt_ref[...] = _l2norm(yt[:, OUT:])
    yb = jnp.dot(_dequant(adjq_ref), z,
                 preferred_element_type=jnp.float32) + b
    oab_ref[...] = _l2norm(yb[:, :OUT])
    obb_ref[...] = _l2norm(yb[:, OUT:])


def _adj_final_call(adj_b, adj_q, inp, w, b):
    width = w.shape[1]
    outs = pl.pallas_call(
        _adj_final_body,
        grid=(NB,),
        in_specs=[_blk(RB), _blk(RB), _full((N, FIN)),
                  _full((FIN, width)), _full((1, width))],
        out_specs=[_out_blk(OUT)] * 4,
        out_shape=[jax.ShapeDtypeStruct((NTOP, OUT), jnp.float32)] * 4,
        scratch_shapes=[pltpu.VMEM((N, width), jnp.bfloat16)],
    )(adj_b, adj_q, inp, w, b)
    oat, obt, oab, obb = outs
    return (jnp.concatenate([oat, oab], axis=0),
            jnp.concatenate([obt, obb], axis=0))


def _snowball_pair(x_pad, adj, pa, pb):
    inp = x_pad
    adj_b = adj_q = None
    for k in range(NLAYERS):
        wc = _combine_w(pa["Ws"][k], pb["Ws"][k], k)
        bc = jnp.concatenate([pa["bs"][k], pb["bs"][k]]).reshape(1, 2 * NHID)
        if k == 0:
            ht, hb, adj_b, adj_q = _adj_first_call(adj, inp, wc, bc)
        else:
            ht, hb = _adj_mid_call(adj_b, adj_q, inp, wc, bc)
        col = NFEAT + 2 * NHID * k
        inp = jax.lax.dynamic_update_slice(inp, ht, (0, col))
        inp = jax.lax.dynamic_update_slice(inp, hb, (NTOP, col))
    wc = _combine_w(pa["Wout"], pb["Wout"], NLAYERS)
    bc = jnp.concatenate([pa["bout"], pb["bout"]]).reshape(1, 2 * OUT)
    return _adj_final_call(adj_b, adj_q, inp, wc, bc)


def _att_body(e1_ref, e2_ref, e3_ref, c1_ref, c2_ref, c3_ref,
              w1_ref, b1_ref, w2_ref, mw_ref, mb_ref, out_ref, beta_ref):
    xcom = (c1_ref[...] + c2_ref[...] + c3_ref[...]) / 3.0
    embs = (e1_ref[...], e2_ref[...], e3_ref[...], xcom)
    w2 = w2_ref[...]
    cols = []
    for e in embs:
        t = jnp.tanh(jnp.dot(e, w1_ref[...],
                             preferred_element_type=jnp.float32) + b1_ref[...])
        cols.append(t[:, 0:1] * w2[0:1, 0:1] + t[:, 1:2] * w2[1:2, 0:1])
    w = jnp.concatenate(cols, axis=1)
    m = jnp.max(w, axis=1, keepdims=True)
    ew = jnp.exp(w - m)
    beta = ew / jnp.sum(ew, axis=1, keepdims=True)
    beta_ref[...] = beta
    emb_att = (beta[:, 0:1] * embs[0] + beta[:, 1:2] * embs[1]
               + beta[:, 2:3] * embs[2] + beta[:, 3:4] * embs[3])
    logits = jnp.dot(emb_att, mw_ref[...],
                     preferred_element_type=jnp.float32) + mb_ref[...]
    mm = jnp.max(logits, axis=1, keepdims=True)
    el = jnp.exp(logits - mm)
    out_ref[...] = el / jnp.sum(el, axis=1, keepdims=True)


def _att_call(e1, e2, e3, c1, c2, c3, att_w1, att_b1, att_w2, mlp_w, mlp_b):
    emb_spec = pl.BlockSpec((RB_ATT, OUT), lambda i: (i, 0))
    return pl.pallas_call(
        _att_body,
        grid=(N // RB_ATT,),
        in_specs=[
            emb_spec, emb_spec, emb_spec, emb_spec, emb_spec, emb_spec,
            _full((OUT, 2)), _full((1, 2)), _full((2, 1)),
            _full((OUT, NCLASS)), _full((1, NCLASS)),
        ],
        out_specs=[
            pl.BlockSpec((RB_ATT, NCLASS), lambda i: (i, 0)),
            pl.BlockSpec((RB_ATT, 4), lambda i: (i, 0)),
        ],
        out_shape=[
            jax.ShapeDtypeStruct((N, NCLASS), jnp.float32),
            jax.ShapeDtypeStruct((N, 4), jnp.float32),
        ],
    )(e1, e2, e3, c1, c2, c3, att_w1, att_b1.reshape(1, 2), att_w2,
      mlp_w, mlp_b.reshape(1, NCLASS))


def kernel(x, sadj, fadj, fadj2, sgcn1, sgcn2, sgcn3, cgcn,
           att_w1, att_b1, att_w2, mlp_w, mlp_b):
    x_pad = jnp.pad(x.astype(jnp.bfloat16), ((0, 0), (0, FIN - NFEAT)))
    emb1, com1 = _snowball_pair(x_pad, sadj, sgcn1, cgcn)
    emb2, com2 = _snowball_pair(x_pad, fadj, sgcn2, cgcn)
    emb3, com3 = _snowball_pair(x_pad, fadj2, sgcn3, cgcn)
    output, beta4 = _att_call(emb1, emb2, emb3, com1, com2, com3,
                              att_w1, att_b1, att_w2, mlp_w, mlp_b)
    beta = beta4.reshape(N, 4, 1)
    return (output, beta, emb1, com1, com2, com3, emb2, emb3)


# speedup vs baseline: 1.2728x; 1.2169x over previous
import jax
import jax.numpy as jnp
from jax.experimental import pallas as pl
from jax.experimental.pallas import tpu as pltpu

N = 10000
NFEAT = 128
NHID = 16
OUT = 64
NCLASS = 10
NLAYERS = 9

NB = 25
RB = N // NB
NMEGA = NLAYERS
W2 = 2 * OUT
FINP = 128 * (NLAYERS + 1)
RB_ATT = 1000


def _combine_w(wa, wb, k):
    ca, cb = wa.shape[1], wb.shape[1]
    parts = [jnp.concatenate([wa[:NFEAT], wb[:NFEAT]], axis=1)]
    for j in range(k):
        ra = wa[NFEAT + NHID * j:NFEAT + NHID * (j + 1)]
        rb = wb[NFEAT + NHID * j:NFEAT + NHID * (j + 1)]
        parts.append(jnp.concatenate(
            [ra, jnp.zeros((NHID, cb), jnp.float32)], axis=1))
        parts.append(jnp.concatenate(
            [jnp.zeros((NHID, ca), jnp.float32), rb], axis=1))
        parts.append(jnp.zeros((128 - 2 * NHID, ca + cb), jnp.float32))
    rows = NFEAT + 128 * k
    parts.append(jnp.zeros((FINP - rows, ca + cb), jnp.float32))
    return jnp.concatenate(parts, axis=0)


def _l2norm(y):
    n = jnp.maximum(jnp.sqrt(jnp.sum(y * y, axis=1, keepdims=True)), 1e-12)
    return y / n


def _adj_first_body(adj_ref, x_ref, w_ref, b_ref, h_ref, adjb_ref, xw_ref):
    @pl.when(pl.program_id(0) == 0)
    def _():
        xw_ref[...] = jnp.dot(x_ref[...], w_ref[...],
                              preferred_element_type=jnp.float32
                              ).astype(jnp.bfloat16)

    ab = adj_ref[...].astype(jnp.bfloat16)
    adjb_ref[...] = ab
    y = jnp.dot(ab, xw_ref[...], preferred_element_type=jnp.float32)
    h_ref[...] = jnp.tanh(y + b_ref[...]).astype(jnp.bfloat16)


def _adj_first_call(adj, x_b, w0, b0):
    width = w0.shape[1]
    return pl.pallas_call(
        _adj_first_body,
        grid=(NB,),
        in_specs=[
            pl.BlockSpec((RB, N), lambda i: (i, 0)),
            pl.BlockSpec((N, NFEAT), lambda i: (0, 0)),
            pl.BlockSpec((NFEAT, width), lambda i: (0, 0)),
            pl.BlockSpec((1, width), lambda i: (0, 0)),
        ],
        out_specs=[
            pl.BlockSpec((RB, width), lambda i: (i, 0)),
            pl.BlockSpec((RB, N), lambda i: (i, 0)),
        ],
        out_shape=[
            jax.ShapeDtypeStruct((N, width), jnp.bfloat16),
            jax.ShapeDtypeStruct((N, N), jnp.bfloat16),
        ],
        scratch_shapes=[pltpu.VMEM((N, width), jnp.bfloat16)],
    )(adj, x_b, w0, b0)


def _mega_body(adjb_ref, x_ref, h_ref, w_ref, b_ref,
               oa_ref, ob_ref, inp_s, xw_s):
    kk = pl.program_id(0)
    i = pl.program_id(1)

    @pl.when((kk == 0) & (i == 0))
    def _():
        inp_s[...] = jnp.zeros((N, FINP), jnp.bfloat16)
        inp_s[:, :NFEAT] = x_ref[...]
        inp_s[:, NFEAT:NFEAT + 2 * NHID] = h_ref[...]

    @pl.when(i == 0)
    def _():
        xw_s[...] = jnp.dot(inp_s[...], w_ref[0],
                            preferred_element_type=jnp.float32
                            ).astype(jnp.bfloat16)

    y = jnp.dot(adjb_ref[...], xw_s[...],
                preferred_element_type=jnp.float32) + b_ref[0]

    @pl.when(kk < NMEGA - 1)
    def _():
        col = 128 * (kk + 2)
        z = jnp.zeros((RB, 128 - 2 * NHID), jnp.bfloat16)
        h = jnp.concatenate(
            [jnp.tanh(y[:, :2 * NHID]).astype(jnp.bfloat16), z], axis=1)
        inp_s[pl.ds(i * RB, RB), pl.ds(col, 128)] = h

    @pl.when(kk == NMEGA - 1)
    def _():
        oa_ref[...] = _l2norm(y[:, :OUT])
        ob_ref[...] = _l2norm(y[:, OUT:])


def _mega_call(adj_b, x_b, h0, w_all, b_all):
    out_spec = pl.BlockSpec((RB, OUT),
                            lambda k, i: ((k // (NMEGA - 1)) * i, 0))
    return pl.pallas_call(
        _mega_body,
        grid=(NMEGA, NB),
        in_specs=[
            pl.BlockSpec((RB, N), lambda k, i: (i, 0)),
            pl.BlockSpec((N, NFEAT), lambda k, i: (0, 0)),
            pl.BlockSpec((N, 2 * NHID), lambda k, i: (0, 0)),
            pl.BlockSpec((1, FINP, W2), lambda k, i: (k, 0, 0)),
            pl.BlockSpec((1, 1, W2), lambda k, i: (k, 0, 0)),
        ],
        out_specs=[out_spec, out_spec],
        out_shape=[
            jax.ShapeDtypeStruct((N, OUT), jnp.float32),
            jax.ShapeDtypeStruct((N, OUT), jnp.float32),
        ],
        scratch_shapes=[pltpu.VMEM((N, FINP), jnp.bfloat16),
                        pltpu.VMEM((N, W2), jnp.bfloat16)],
    )(adj_b, x_b, h0, w_all, b_all)


def _snowball_pair(x_b, adj, pa, pb):
    w0 = jnp.concatenate([pa["Ws"][0], pb["Ws"][0]], axis=1
                         ).astype(jnp.bfloat16)
    b0 = jnp.concatenate([pa["bs"][0], pb["bs"][0]]).reshape(1, 2 * NHID)
    h0, adj_b = _adj_first_call(adj, x_b, w0, b0)
    ws, bs = [], []
    for k in range(1, NLAYERS):
        w = _combine_w(pa["Ws"][k], pb["Ws"][k], k)
        ws.append(jnp.pad(w, ((0, 0), (0, W2 - w.shape[1]))))
        bk = jnp.concatenate([pa["bs"][k], pb["bs"][k]])
        bs.append(jnp.pad(bk, (0, W2 - bk.shape[0])).reshape(1, W2))
    ws.append(_combine_w(pa["Wout"], pb["Wout"], NLAYERS))
    bs.append(jnp.concatenate([pa["bout"], pb["bout"]]).reshape(1, W2))
    w_all = jnp.stack(ws).astype(jnp.bfloat16)
    b_all = jnp.stack(bs)
    return _mega_call(adj_b, x_b, h0, w_all, b_all)


def _att_body(e1_ref, e2_ref, e3_ref, c1_ref, c2_ref, c3_ref,
              w1_ref, b1_ref, w2_ref, mw_ref, mb_ref, out_ref, beta_ref):
    xcom = (c1_ref[...] + c2_ref[...] + c3_ref[...]) / 3.0
    embs = (e1_ref[...], e2_ref[...], e3_ref[...], xcom)
    w2 = w2_ref[...]
    cols = []
    for e in embs:
        t = jnp.tanh(jnp.dot(e, w1_ref[...],
                             preferred_element_type=jnp.float32) + b1_ref[...])
        cols.append(t[:, 0:1] * w2[0:1, 0:1] + t[:, 1:2] * w2[1:2, 0:1])
    w = jnp.concatenate(cols, axis=1)
    m = jnp.max(w, axis=1, keepdims=True)
    ew = jnp.exp(w - m)
    beta = ew / jnp.sum(ew, axis=1, keepdims=True)
    beta_ref[...] = beta
    emb_att = (beta[:, 0:1] * embs[0] + beta[:, 1:2] * embs[1]
               + beta[:, 2:3] * embs[2] + beta[:, 3:4] * embs[3])
    logits = jnp.dot(emb_att, mw_ref[...],
                     preferred_element_type=jnp.float32) + mb_ref[...]
    mm = jnp.max(logits, axis=1, keepdims=True)
    el = jnp.exp(logits - mm)
    out_ref[...] = el / jnp.sum(el, axis=1, keepdims=True)


def _att_call(e1, e2, e3, c1, c2, c3, att_w1, att_b1, att_w2, mlp_w, mlp_b):
    emb_spec = pl.BlockSpec((RB_ATT, OUT), lambda i: (i, 0))
    full = lambda shape: pl.BlockSpec(shape, lambda i: (0, 0))
    return pl.pallas_call(
        _att_body,
        grid=(N // RB_ATT,),
        in_specs=[
            emb_spec, emb_spec, emb_spec, emb_spec, emb_spec, emb_spec,
            full((OUT, 2)), full((1, 2)), full((2, 1)),
            full((OUT, NCLASS)), full((1, NCLASS)),
        ],
        out_specs=[
            pl.BlockSpec((RB_ATT, NCLASS), lambda i: (i, 0)),
            pl.BlockSpec((RB_ATT, 4), lambda i: (i, 0)),
        ],
        out_shape=[
            jax.ShapeDtypeStruct((N, NCLASS), jnp.float32),
            jax.ShapeDtypeStruct((N, 4), jnp.float32),
        ],
    )(e1, e2, e3, c1, c2, c3, att_w1, att_b1.reshape(1, 2), att_w2,
      mlp_w, mlp_b.reshape(1, NCLASS))


def kernel(x, sadj, fadj, fadj2, sgcn1, sgcn2, sgcn3, cgcn,
           att_w1, att_b1, att_w2, mlp_w, mlp_b):
    x_b = x.astype(jnp.bfloat16)
    emb1, com1 = _snowball_pair(x_b, sadj, sgcn1, cgcn)
    emb2, com2 = _snowball_pair(x_b, fadj, sgcn2, cgcn)
    emb3, com3 = _snowball_pair(x_b, fadj2, sgcn3, cgcn)
    output, beta4 = _att_call(emb1, emb2, emb3, com1, com2, com3,
                              att_w1, att_b1, att_w2, mlp_w, mlp_b)
    beta = beta4.reshape(N, 4, 1)
    return (output, beta, emb1, com1, com2, com3, emb2, emb3)
